# baseline (speedup 1.0000x reference)
import numpy as np
import jax
import jax.numpy as jnp
from jax.experimental import pallas as pl


def _pool_matrix(H, W, OH, OW):
    bh, bw = H // OH, W // OW
    P = np.zeros((H * W, OH * OW), np.float32)
    for h in range(H):
        for w in range(W):
            P[h * W + w, (h // bh) * OW + (w // bw)] = 1.0 / (bh * bw)
    return P


def _pool_body(x_ref, p_ref, o_ref):
    blk = x_ref.shape[0]
    xr = x_ref[...].reshape(blk, x_ref.shape[1] * x_ref.shape[2])
    o_ref[...] = jnp.dot(xr, p_ref[...], preferred_element_type=jnp.float32)


def _head_body(f_ref, w1_ref, b1_ref, w2_ref, b2_ref, g_ref, i_ref):
    h = jax.lax.dot_general(f_ref[...], w1_ref[...],
                            (((1,), (1,)), ((), ())),
                            preferred_element_type=jnp.float32) + b1_ref[...]
    h = jnp.maximum(h, 0.0)
    logits = jax.lax.dot_general(h, w2_ref[...],
                                 (((1,), (1,)), ((), ())),
                                 preferred_element_type=jnp.float32) + b2_ref[...]
    B, E = logits.shape
    lane = jax.lax.broadcasted_iota(jnp.int32, (B, E), 1)
    m1 = jnp.max(logits, axis=-1, keepdims=True)
    i1 = jnp.min(jnp.where(logits == m1, lane, E), axis=-1, keepdims=True)
    masked = jnp.where(lane == i1, -jnp.inf, logits)
    m2 = jnp.max(masked, axis=-1, keepdims=True)
    i2 = jnp.min(jnp.where(masked == m2, lane, E), axis=-1, keepdims=True)
    e2 = jnp.exp(m2 - m1)
    g1 = 1.0 / (1.0 + e2)
    g2 = e2 / (1.0 + e2)
    g_ref[...] = (jnp.where(lane == i1, g1, 0.0)
                  + jnp.where(lane == i2, g2, 0.0))
    i_ref[...] = jnp.where(lane == 0, i1, 0) + jnp.where(lane == 1, i2, 0)


def kernel(x, W1, b1, W2, b2):
    B, C, H, W = x.shape
    E = W2.shape[0]
    OH = OW = 4
    S = H * W
    x3 = x.reshape(B * C, H, W)
    P = jnp.asarray(_pool_matrix(H, W, OH, OW))
    ROWS = B * C
    BLK = 512
    pooled = pl.pallas_call(
        _pool_body,
        grid=(ROWS // BLK,),
        in_specs=[pl.BlockSpec((BLK, H, W), lambda i: (i, 0, 0)),
                  pl.BlockSpec((S, OH * OW), lambda i: (0, 0))],
        out_specs=pl.BlockSpec((BLK, OH * OW), lambda i: (i, 0)),
        out_shape=jax.ShapeDtypeStruct((ROWS, OH * OW), jnp.float32),
    )(x3, P)
    flat = pooled.reshape(B, C * OH * OW)
    gates, ipad = pl.pallas_call(
        _head_body,
        out_shape=[jax.ShapeDtypeStruct((B, E), jnp.float32),
                   jax.ShapeDtypeStruct((B, E), jnp.int32)],
    )(flat, W1, b1.reshape(1, -1), W2, b2.reshape(1, -1))
    return gates, ipad[:, :2]


# speedup vs baseline: 1.0848x; 1.0848x over previous
import numpy as np
import jax
import jax.numpy as jnp
from jax.experimental import pallas as pl


def _pool_matrix(H, W, OH, OW):
    bh, bw = H // OH, W // OW
    P = np.zeros((H * W, OH * OW), np.float32)
    for h in range(H):
        for w in range(W):
            P[h * W + w, (h // bh) * OW + (w // bw)] = 1.0 / (bh * bw)
    return P


def _pool_body(x_ref, p_ref, o_ref):
    o_ref[...] = jnp.dot(x_ref[...], p_ref[...],
                         preferred_element_type=jnp.float32)


def _head_body(f_ref, w1_ref, b1_ref, w2_ref, b2_ref, g_ref, i_ref):
    h = jax.lax.dot_general(f_ref[...], w1_ref[...],
                            (((1,), (1,)), ((), ())),
                            preferred_element_type=jnp.float32) + b1_ref[...]
    h = jnp.maximum(h, 0.0)
    logits = jax.lax.dot_general(h, w2_ref[...],
                                 (((1,), (1,)), ((), ())),
                                 preferred_element_type=jnp.float32) + b2_ref[...]
    B, E = logits.shape
    lane = jax.lax.broadcasted_iota(jnp.int32, (B, E), 1)
    m1 = jnp.max(logits, axis=-1, keepdims=True)
    i1 = jnp.min(jnp.where(logits == m1, lane, E), axis=-1, keepdims=True)
    masked = jnp.where(lane == i1, -jnp.inf, logits)
    m2 = jnp.max(masked, axis=-1, keepdims=True)
    i2 = jnp.min(jnp.where(masked == m2, lane, E), axis=-1, keepdims=True)
    e2 = jnp.exp(m2 - m1)
    g1 = 1.0 / (1.0 + e2)
    g2 = e2 / (1.0 + e2)
    g_ref[...] = (jnp.where(lane == i1, g1, 0.0)
                  + jnp.where(lane == i2, g2, 0.0))
    i_ref[...] = jnp.where(lane == 0, i1, 0) + jnp.where(lane == 1, i2, 0)


def kernel(x, W1, b1, W2, b2):
    B, C, H, W = x.shape
    E = W2.shape[0]
    OH = OW = 4
    S = H * W
    x2d = x.reshape(B * C, S)
    P = jnp.asarray(_pool_matrix(H, W, OH, OW))
    ROWS = B * C
    BLK = 2048
    pooled = pl.pallas_call(
        _pool_body,
        grid=(ROWS // BLK,),
        in_specs=[pl.BlockSpec((BLK, S), lambda i: (i, 0)),
                  pl.BlockSpec((S, OH * OW), lambda i: (0, 0))],
        out_specs=pl.BlockSpec((BLK, OH * OW), lambda i: (i, 0)),
        out_shape=jax.ShapeDtypeStruct((ROWS, OH * OW), jnp.float32),
    )(x2d, P)
    flat = pooled.reshape(B, C * OH * OW)
    gates, ipad = pl.pallas_call(
        _head_body,
        out_shape=[jax.ShapeDtypeStruct((B, E), jnp.float32),
                   jax.ShapeDtypeStruct((B, E), jnp.int32)],
    )(flat, W1, b1.reshape(1, -1), W2, b2.reshape(1, -1))
    return gates, ipad[:, :2]
